# baseline (speedup 1.0000x reference)
import functools

import jax
import jax.numpy as jnp
from jax import lax
from jax.experimental import pallas as pl
from jax.experimental.pallas import tpu as pltpu
from jax.experimental.pallas import tpu_sc as plsc

L = 16


@functools.lru_cache(maxsize=None)
def _build(B, D):
    info = plsc.get_sparse_core_info()
    NC, NS = info.num_cores, info.num_subcores
    NW = NC * NS
    assert B % (8 * NW) == 0 and D % L == 0
    bpw = B // NW

    mesh = plsc.VectorSubcoreMesh(core_axis_name="c", subcore_axis_name="s")

    @functools.partial(
        pl.kernel,
        mesh=mesh,
        compiler_params=pltpu.CompilerParams(
            needs_layout_passes=False, use_tc_tiling_on_sc=False),
        out_type=(
            jax.ShapeDtypeStruct((B,), jnp.float32),
            jax.ShapeDtypeStruct((B,), jnp.float32),
        ),
        scratch_types=[
            pltpu.VMEM((bpw,), jnp.int32),
            pltpu.VMEM((bpw,), jnp.int32),
            pltpu.VMEM((bpw, D), jnp.float32),
            pltpu.VMEM((bpw, D), jnp.float32),
            pltpu.VMEM((bpw,), jnp.float32),
            pltpu.VMEM((bpw,), jnp.float32),
            pltpu.VMEM((bpw,), jnp.float32),
            pltpu.VMEM((bpw,), jnp.float32),
            pltpu.VMEM((2 * L,), jnp.float32),
            pltpu.SemaphoreType.DMA,
            pltpu.SemaphoreType.DMA,
            pltpu.SemaphoreType.DMA,
            pltpu.SemaphoreType.DMA,
        ],
    )
    def k(users, items, user_emb, item_emb, user_b, prod_b, scal,
          pred_out, log_out,
          u2_v, it_v, ur_v, ir_v, ub_v, pb_v, log_v, pred_v, sc_v,
          s0, s1, s2, s3):
        wid = lax.axis_index("s") * NC + lax.axis_index("c")
        base = wid * bpw

        pltpu.sync_copy(users.at[pl.ds(base, bpw)], u2_v)
        pltpu.sync_copy(items.at[pl.ds(base, bpw)], it_v)
        pltpu.sync_copy(scal, sc_v)

        def _dbl(j, carry):
            sl = pl.ds(j * L, L)
            v = u2_v[sl]
            u2_v[sl] = v + v
            return carry

        lax.fori_loop(0, bpw // L, _dbl, 0)

        cp0 = pltpu.async_copy(user_emb.at[u2_v], ur_v, s0)
        cp1 = pltpu.async_copy(item_emb.at[it_v], ir_v, s1)
        cp2 = pltpu.async_copy(user_b.at[u2_v], ub_v, s2)
        cp3 = pltpu.async_copy(prod_b.at[it_v], pb_v, s3)
        cp0.wait()
        cp1.wait()
        cp2.wait()
        cp3.wait()

        alpha_s = sc_v[pl.ds(0, L)]
        g_s = sc_v[pl.ds(L, L)]

        def _blk(b, carry):
            off = b * L
            rows = off + lax.iota(jnp.int32, L)
            acc = jnp.zeros((L,), jnp.float32)
            for dd in range(D):
                cols = jnp.full((L,), dd, jnp.int32)
                u = plsc.load_gather(ur_v, [rows, cols])
                iv = plsc.load_gather(ir_v, [rows, cols])
                acc = acc + u * iv
            sl = pl.ds(off, L)
            logit = alpha_s * acc + ub_v[sl] + pb_v[sl] + g_s
            log_v[sl] = logit
            pred_v[sl] = 1.0 / (1.0 + jnp.exp(-logit))
            return carry

        lax.fori_loop(0, bpw // L, _blk, 0)

        pltpu.sync_copy(log_v, log_out.at[pl.ds(base, bpw)])
        pltpu.sync_copy(pred_v, pred_out.at[pl.ds(base, bpw)])

    return k


def kernel(users, items, user_emb, item_emb, alpha, global_bias, user_b, prod_b):
    B = users.shape[0]
    D = user_emb.shape[1]
    users = users.astype(jnp.int32)
    items = items.astype(jnp.int32)
    scal = jnp.concatenate([
        jnp.broadcast_to(alpha.astype(jnp.float32), (L,)),
        jnp.broadcast_to(global_bias.astype(jnp.float32), (L,)),
    ])
    pred, logits = _build(B, D)(users, items, user_emb, item_emb,
                                user_b, prod_b, scal)
    return pred.reshape(B, 1), logits.reshape(B, 1)
